# baseline (speedup 1.0000x reference)
import functools

import jax
import jax.numpy as jnp
from jax import lax
from jax.experimental import pallas as pl
from jax.experimental.pallas import tpu as pltpu
from jax.experimental.pallas import tpu_sc as plsc

BATCH = 16384
D = 32
NC = 2
NS = 16
NW = NC * NS
B_PER_W = BATCH // NW


def _sc_gather4(u_mlp_tab, i_mlp_tab, u_mf_tab, i_mf_tab, uidx, iidx):
  mesh = plsc.VectorSubcoreMesh(core_axis_name="c", subcore_axis_name="s")
  row = jax.ShapeDtypeStruct((BATCH, D), jnp.float32)

  @functools.partial(
      pl.kernel,
      mesh=mesh,
      out_type=[row, row, row, row],
      compiler_params=pltpu.CompilerParams(use_tc_tiling_on_sc=False),
      scratch_types=[
          pltpu.VMEM((B_PER_W,), jnp.int32),
          pltpu.VMEM((B_PER_W,), jnp.int32),
          pltpu.VMEM((B_PER_W, D), jnp.float32),
          pltpu.VMEM((B_PER_W, D), jnp.float32),
          pltpu.VMEM((B_PER_W, D), jnp.float32),
          pltpu.VMEM((B_PER_W, D), jnp.float32),
          pltpu.SemaphoreType.DMA,
          pltpu.SemaphoreType.DMA,
      ],
  )
  def k(ut_hbm, it_hbm, umf_hbm, imf_hbm, ui_hbm, ii_hbm,
        o1, o2, o3, o4, ui_v, ii_v, r1, r2, r3, r4, gsem, osem):
    wid = lax.axis_index("s") * NC + lax.axis_index("c")
    base = wid * B_PER_W
    pltpu.sync_copy(ui_hbm.at[pl.ds(base, B_PER_W)], ui_v)
    pltpu.sync_copy(ii_hbm.at[pl.ds(base, B_PER_W)], ii_v)
    c1 = pltpu.async_copy(ut_hbm.at[ui_v], r1, gsem)
    c2 = pltpu.async_copy(it_hbm.at[ii_v], r2, gsem)
    c3 = pltpu.async_copy(umf_hbm.at[ui_v], r3, gsem)
    c4 = pltpu.async_copy(imf_hbm.at[ii_v], r4, gsem)
    c1.wait()
    w1 = pltpu.async_copy(r1, o1.at[pl.ds(base, B_PER_W)], osem)
    c2.wait()
    w2 = pltpu.async_copy(r2, o2.at[pl.ds(base, B_PER_W)], osem)
    c3.wait()
    w3 = pltpu.async_copy(r3, o3.at[pl.ds(base, B_PER_W)], osem)
    c4.wait()
    w4 = pltpu.async_copy(r4, o4.at[pl.ds(base, B_PER_W)], osem)
    w1.wait()
    w2.wait()
    w3.wait()
    w4.wait()

  return k(u_mlp_tab, i_mlp_tab, u_mf_tab, i_mf_tab, uidx, iidx)


def _tc_mlp(gu_mlp, gi_mlp, gu_mf, gi_mf, W1a, W1b, b1, W2t, b2, W3t, b3,
            wo_mlp, wo_mf, bo):
  blk = 4096
  grid = (BATCH // blk,)

  def body(u_ref, i_ref, umf_ref, imf_ref, w1a_ref, w1b_ref, b1_ref,
           w2_ref, b2_ref, w3_ref, b3_ref, womlp_ref, womf_ref, bo_ref,
           o_ref):
    u = u_ref[...]
    it = i_ref[...]
    h = jnp.dot(u, w1a_ref[...], preferred_element_type=jnp.float32)
    h += jnp.dot(it, w1b_ref[...], preferred_element_type=jnp.float32)
    h = jnp.maximum(h + b1_ref[...], 0.0)
    h = jnp.dot(h, w2_ref[...], preferred_element_type=jnp.float32)
    h = jnp.maximum(h + b2_ref[...], 0.0)
    h = jnp.dot(h, w3_ref[...], preferred_element_type=jnp.float32)
    h = jnp.maximum(h + b3_ref[...], 0.0)
    mf = umf_ref[...] * imf_ref[...]
    logit = jnp.dot(h, womlp_ref[...], preferred_element_type=jnp.float32)
    logit += jnp.dot(mf, womf_ref[...], preferred_element_type=jnp.float32)
    o_ref[...] = jax.nn.sigmoid(logit + bo_ref[...])

  rows = pl.BlockSpec((blk, D), lambda i: (i, 0))
  full = lambda s: pl.BlockSpec(s, lambda i: tuple(0 for _ in s))
  return pl.pallas_call(
      body,
      grid=grid,
      in_specs=[
          rows, rows, rows, rows,
          full((D, D)), full((D, D)), full((1, D)),
          full((D, 16)), full((1, 16)),
          full((16, 8)), full((1, 8)),
          full((8, 1)), full((D, 1)), full((1, 1)),
      ],
      out_specs=pl.BlockSpec((blk, 1), lambda i: (i, 0)),
      out_shape=jax.ShapeDtypeStruct((BATCH, 1), jnp.float32),
  )(gu_mlp, gi_mlp, gu_mf, gi_mf, W1a, W1b, b1, W2t, b2, W3t, b3,
    wo_mlp, wo_mf, bo)


def kernel(user_indices, item_indices, emb_user_mlp, emb_item_mlp,
           emb_user_mf, emb_item_mf, W1, b1, W2, b2, W3, b3, Wo, bo):
  uidx = user_indices.astype(jnp.int32)
  iidx = item_indices.astype(jnp.int32)

  gu_mlp, gi_mlp, gu_mf, gi_mf = _sc_gather4(
      emb_user_mlp, emb_item_mlp, emb_user_mf, emb_item_mf, uidx, iidx)

  W1a = W1[:, :D].T
  W1b = W1[:, D:].T
  W2t = W2.T
  W3t = W3.T
  wo_mlp = Wo[:, :8].T
  wo_mf = Wo[:, 8:].T

  out = _tc_mlp(gu_mlp, gi_mlp, gu_mf, gi_mf,
                W1a, W1b, b1.reshape(1, -1), W2t, b2.reshape(1, -1),
                W3t, b3.reshape(1, -1), wo_mlp, wo_mf, bo.reshape(1, 1))
  return out.reshape(BATCH)
